# Initial kernel scaffold; baseline (speedup 1.0000x reference)
import jax
import jax.numpy as jnp
from jax.experimental import pallas as pl


def kernel(x, in_degree, out_degree, batch, in_table, out_table, graph_token):
    raise NotImplementedError("write your pallas kernel here")


# speedup vs baseline: 2.7179x; 2.7179x over previous
import functools

import jax
import jax.numpy as jnp
from jax import lax
from jax.experimental import pallas as pl
from jax.experimental.pallas import tpu as pltpu
from jax.experimental.pallas import tpu_sc as plsc

N, D, B, DEG_VOCAB = 100000, 128, 16, 65
L = 16
C = 128
NC, NS = 2, 16
NW = NC * NS
NT_FULL = N // C
TAIL = N - NT_FULL * C
TAIL_BASE = NT_FULL * C
B2_CHUNK = 3136
B2_LAST = (N + B) - (NW - 1) * B2_CHUNK


def _sc_encode(x_hbm, din_hbm, dout_hbm, starts_hbm, tin_hbm, tout_hbm,
               gt_hbm, outh_hbm, outb_hbm,
               starts_v, h_v, ein_v, eout_v, din_v, dout_v, off_v,
               gt_v, tok_v, tokidx_v, b2_v, dint_v, doutt_v, offt_v,
               sem0, sem1):
    wid = lax.axis_index("s") * NC + lax.axis_index("c")
    iota = lax.iota(jnp.int32, L)

    pltpu.sync_copy(starts_hbm, starts_v)
    _sv = starts_v[...]
    splats = [_sv.at[jnp.full((L,), g, jnp.int32)].get(mode="promise_in_bounds")
              for g in range(B)]
    ones = jnp.full((L,), 1, jnp.int32)
    zeros = jnp.full((L,), 0, jnp.int32)

    def count_starts_le(i_v):
        cnt = jnp.zeros((L,), jnp.int32)
        for g in range(B):
            cnt = cnt + jnp.where(i_v >= splats[g], ones, zeros)
        return cnt

    def add_rows(nrows):
        def row(r, _):
            for c8 in range(D // L):
                s = pl.ds(c8 * L, L)
                h_v[r, s] = h_v[r, s] + ein_v[r, s] + eout_v[r, s]
            return 0
        lax.fori_loop(0, nrows, row, 0)

    nt = (NT_FULL - 1 - wid) // NW + 1

    def tile_body(k, _):
        base = (wid + k * NW) * C
        pltpu.sync_copy(x_hbm.at[pl.ds(base, C)], h_v)
        pltpu.sync_copy(din_hbm.at[pl.ds(base, C)], din_v)
        pltpu.sync_copy(dout_hbm.at[pl.ds(base, C)], dout_v)
        cin = pltpu.async_copy(tin_hbm.at[din_v], ein_v, sem0)
        cot = pltpu.async_copy(tout_hbm.at[dout_v], eout_v, sem1)
        cin.wait()
        cot.wait()

        def og(j, _):
            i_v = base + j * L + iota
            off_v[pl.ds(j * L, L)] = i_v + count_starts_le(i_v)
            return 0
        lax.fori_loop(0, C // L, og, 0)
        add_rows(C)
        pltpu.async_copy(h_v, outh_hbm.at[off_v], sem0).wait()
        return 0

    lax.fori_loop(0, nt, tile_body, 0)

    @pl.when(wid == 0)
    def _():
        pltpu.sync_copy(x_hbm.at[pl.ds(TAIL_BASE, TAIL)], h_v.at[pl.ds(0, TAIL)])
        pltpu.sync_copy(din_hbm.at[pl.ds(TAIL_BASE, TAIL)], dint_v)
        pltpu.sync_copy(dout_hbm.at[pl.ds(TAIL_BASE, TAIL)], doutt_v)
        pltpu.async_copy(tin_hbm.at[dint_v], ein_v.at[pl.ds(0, TAIL)], sem0).wait()
        pltpu.async_copy(tout_hbm.at[doutt_v], eout_v.at[pl.ds(0, TAIL)], sem0).wait()

        def ogt(j, _):
            i_v = TAIL_BASE + j * L + iota
            offt_v[pl.ds(j * L, L)] = i_v + count_starts_le(i_v)
            return 0
        lax.fori_loop(0, TAIL // L, ogt, 0)
        add_rows(TAIL)
        pltpu.async_copy(h_v.at[pl.ds(0, TAIL)], outh_hbm.at[offt_v], sem0).wait()

        pltpu.sync_copy(gt_hbm, gt_v)
        tokidx_v[...] = starts_v[...] + iota

        def trow(r, _):
            for c8 in range(D // L):
                s = pl.ds(c8 * L, L)
                tok_v[r, s] = gt_v[0, s]
            return 0
        lax.fori_loop(0, B, trow, 0)
        pltpu.async_copy(tok_v, outh_hbm.at[tokidx_v], sem0).wait()

    toks = [splats[g] + g for g in range(B)]
    b2base = wid * B2_CHUNK

    def bg(j, _):
        j_v = b2base + j * L + iota
        cnt = jnp.zeros((L,), jnp.int32)
        for g in range(B):
            cnt = cnt + jnp.where(j_v >= toks[g], ones, zeros)
        b2_v[pl.ds(j * L, L)] = cnt - 1
        return 0
    lax.fori_loop(0, B2_CHUNK // L, bg, 0)

    @pl.when(wid < NW - 1)
    def _():
        pltpu.sync_copy(b2_v, outb_hbm.at[pl.ds(b2base, B2_CHUNK)])

    @pl.when(wid == NW - 1)
    def _():
        pltpu.sync_copy(b2_v.at[pl.ds(0, B2_LAST)],
                        outb_hbm.at[pl.ds(b2base, B2_LAST)])


_sc_call = functools.partial(
    pl.kernel,
    mesh=plsc.VectorSubcoreMesh(core_axis_name="c", subcore_axis_name="s"),
    out_type=[
        jax.ShapeDtypeStruct((N + B, D), jnp.float32),
        jax.ShapeDtypeStruct((N + B,), jnp.int32),
    ],
    scratch_types=[
        pltpu.VMEM((B,), jnp.int32),
        pltpu.VMEM((C, D), jnp.float32),
        pltpu.VMEM((C, D), jnp.float32),
        pltpu.VMEM((C, D), jnp.float32),
        pltpu.VMEM((C,), jnp.int32),
        pltpu.VMEM((C,), jnp.int32),
        pltpu.VMEM((C,), jnp.int32),
        pltpu.VMEM((1, D), jnp.float32),
        pltpu.VMEM((B, D), jnp.float32),
        pltpu.VMEM((B,), jnp.int32),
        pltpu.VMEM((B2_CHUNK,), jnp.int32),
        pltpu.VMEM((TAIL,), jnp.int32),
        pltpu.VMEM((TAIL,), jnp.int32),
        pltpu.VMEM((TAIL,), jnp.int32),
        pltpu.SemaphoreType.DMA,
        pltpu.SemaphoreType.DMA,
    ],
)(_sc_encode)


def kernel(x, in_degree, out_degree, batch, in_table, out_table, graph_token):
    starts = jnp.searchsorted(
        batch, jnp.arange(B, dtype=batch.dtype)).astype(jnp.int32)
    out_h, out_b2 = _sc_call(
        x, in_degree.astype(jnp.int32), out_degree.astype(jnp.int32),
        starts, in_table, out_table, graph_token)
    return out_h, out_b2
